# baseline (speedup 1.0000x reference)
import jax
import jax.numpy as jnp
from jax import lax
from jax.experimental import pallas as pl
from jax.experimental.pallas import tpu as pltpu
from jax.experimental.pallas import tpu_sc as plsc

N_NODES = 10000
N_EDGES = 320000
D = 128
DH = D // 2

NC = 2
NS = 16

CHUNK = 64
EDGES_PER_TILE = 20480
E_PAD = NS * EDGES_PER_TILE
PHASES = 10
PCHUNK = 32
NBUF = 4

ACC_ROWS = 10240
XROWS_MAIN = 640
OUT_LAST = N_NODES - 15 * XROWS_MAIN


def _sc_body(xh_hbm, src_hbm, dst_hbm, out_hbm,
             xsp, acc, src_idx, dst_idx, rows0, rows1, rows2, rows3,
             sem0, sem1, sem2, sem3):
    rows = (rows0, rows1, rows2, rows3)
    sems = (sem0, sem1, sem2, sem3)
    c = lax.axis_index("c")
    s = lax.axis_index("s")

    z16 = jnp.zeros((16,), jnp.float32)

    def zb(i, carry):
        for j in range(DH // 16):
            rows0[i, pl.ds(j * 16, 16)] = z16
        return carry

    lax.fori_loop(0, CHUNK, zb, 0)
    arow0 = s * XROWS_MAIN
    for r in range(XROWS_MAIN // CHUNK):
        pltpu.sync_copy(rows0, acc.at[pl.ds(arow0 + r * CHUNK, CHUNK)])

    @pl.when(s < NS - 1)
    def _():
        pltpu.sync_copy(xh_hbm.at[c, pl.ds(arow0, XROWS_MAIN)],
                        xsp.at[pl.ds(arow0, XROWS_MAIN)])

    @pl.when(s == NS - 1)
    def _():
        pltpu.sync_copy(xh_hbm.at[c, pl.ds(arow0, OUT_LAST)],
                        xsp.at[pl.ds(arow0, OUT_LAST)])

    plsc.subcore_barrier()

    def phase(p, carry):
        pltpu.sync_copy(src_hbm.at[s, p], src_idx)
        pltpu.sync_copy(dst_hbm.at[s, p], dst_idx)
        for j in range(NBUF - 1):
            pltpu.async_copy(xsp.at[src_idx.at[j]], rows[j], sems[j])

        def body(k, carry2):
            for b in range(NBUF):
                kk = NBUF * k + b
                pltpu.make_async_copy(
                    xsp.at[src_idx.at[kk]], rows[b], sems[b]).wait()
                nb = (b + 3) % NBUF
                pltpu.async_copy(
                    xsp.at[src_idx.at[kk + 3]], rows[nb], sems[nb])
                pltpu.sync_copy(rows[b], acc.at[dst_idx.at[kk]], add=True)
            return carry2

        lax.fori_loop(0, (PCHUNK - 4) // NBUF, body, 0)
        for kk in range(PCHUNK - 4, PCHUNK):
            b = kk % NBUF
            pltpu.make_async_copy(
                xsp.at[src_idx.at[kk]], rows[b], sems[b]).wait()
            if kk + 3 < PCHUNK:
                nb = (b + 3) % NBUF
                pltpu.async_copy(
                    xsp.at[src_idx.at[kk + 3]], rows[nb], sems[nb])
            pltpu.sync_copy(rows[b], acc.at[dst_idx.at[kk]], add=True)
        return carry

    lax.fori_loop(0, PHASES, phase, 0)

    plsc.subcore_barrier()

    @pl.when(s < NS - 1)
    def _():
        pltpu.sync_copy(acc.at[pl.ds(arow0, XROWS_MAIN)],
                        out_hbm.at[c, pl.ds(arow0, XROWS_MAIN)])

    @pl.when(s == NS - 1)
    def _():
        pltpu.sync_copy(acc.at[pl.ds(arow0, OUT_LAST)],
                        out_hbm.at[c, pl.ds(arow0, OUT_LAST)])


def _sc_scatter(xh, src4, dst4):
    mesh = plsc.VectorSubcoreMesh(core_axis_name="c", subcore_axis_name="s")
    f = pl.kernel(
        _sc_body,
        out_type=jax.ShapeDtypeStruct((2, N_NODES, DH), jnp.float32),
        mesh=mesh,
        scratch_types=[
            pltpu.VMEM_SHARED((N_NODES, DH), jnp.float32),
            pltpu.VMEM_SHARED((ACC_ROWS, DH), jnp.float32),
            pltpu.VMEM((PCHUNK, CHUNK), jnp.int32),
            pltpu.VMEM((PCHUNK, CHUNK), jnp.int32),
            pltpu.VMEM((CHUNK, DH), jnp.float32),
            pltpu.VMEM((CHUNK, DH), jnp.float32),
            pltpu.VMEM((CHUNK, DH), jnp.float32),
            pltpu.VMEM((CHUNK, DH), jnp.float32),
            pltpu.SemaphoreType.DMA,
            pltpu.SemaphoreType.DMA,
            pltpu.SemaphoreType.DMA,
            pltpu.SemaphoreType.DMA,
        ],
    )
    return f(xh, src4, dst4)


def _tc_body(ah, xr, wr0, wr1, wo, bb, o):
    a0 = ah[0]
    a1 = ah[1]
    o[...] = (jnp.dot(a0, wr0[...], preferred_element_type=jnp.float32)
              + jnp.dot(a1, wr1[...], preferred_element_type=jnp.float32)
              + jnp.dot(xr[...], wo[...], preferred_element_type=jnp.float32)
              + bb[...])


def _tc_combine(aggh, x, wrT, woT, b2):
    mb = 1000
    grid = (N_NODES // mb,)
    return pl.pallas_call(
        _tc_body,
        grid=grid,
        in_specs=[
            pl.BlockSpec((2, mb, DH), lambda i: (0, i, 0)),
            pl.BlockSpec((mb, D), lambda i: (i, 0)),
            pl.BlockSpec((DH, D), lambda i: (0, 0)),
            pl.BlockSpec((DH, D), lambda i: (0, 0)),
            pl.BlockSpec((D, D), lambda i: (0, 0)),
            pl.BlockSpec((1, D), lambda i: (0, 0)),
        ],
        out_specs=pl.BlockSpec((mb, D), lambda i: (i, 0)),
        out_shape=jax.ShapeDtypeStruct((N_NODES, D), jnp.float32),
    )(aggh, x, wrT[:DH], wrT[DH:], woT, b2)


def kernel(x, edge_index, W_rel, W_root, b):
    ei = edge_index.astype(jnp.int32)
    pad = E_PAD - N_EDGES
    srcp = jnp.concatenate([ei[0], jnp.zeros((pad,), jnp.int32)])
    dstp = jnp.concatenate([ei[1], jnp.full((pad,), N_NODES, jnp.int32)])
    src4 = srcp.reshape(NS, PHASES, PCHUNK, CHUNK)
    dst4 = dstp.reshape(NS, PHASES, PCHUNK, CHUNK)
    xh = jnp.stack([x[:, :DH], x[:, DH:]])
    aggh = _sc_scatter(xh, src4, dst4)
    return _tc_combine(aggh, x, W_rel.T, W_root.T, b.reshape(1, D))
